# baseline (speedup 1.0000x reference)
import jax
import jax.numpy as jnp
from jax import lax
from jax.experimental import pallas as pl
from jax.experimental.pallas import tpu as pltpu

D_MODEL = 1024
NUM_TILES = 64
TILES_PER_CLUSTER = 8
GRID_SIZE = 64
N_CLUSTERS = NUM_TILES // TILES_PER_CLUSTER
TG = NUM_TILES * GRID_SIZE
N_CHUNKS = 16
CHUNK = TG // N_CHUNKS
TILES_PER_CHUNK = CHUNK // GRID_SIZE

_NEG = -1e30
_F32 = jnp.float32


def _first_argmax(vals, maxv, width):
    col = lax.broadcasted_iota(jnp.int32, vals.shape, 1)
    cand = jnp.where(vals >= maxv, col, jnp.int32(width))
    return jnp.min(cand, axis=1, keepdims=True)


def _body(x_ref, wc_ref, p_ref, k_ref, v_ref, o_ref,
          xh_ref, tidx_ref, gate_ref, yacc_ref, sacc_ref):
    c = pl.program_id(0)

    @pl.when(c == 0)
    def _routing():
        xb = x_ref[...]
        cl = lax.dot_general(xb, wc_ref[...], (((1,), (1,)), ((), ())),
                             preferred_element_type=_F32)
        cmax = jnp.max(cl, axis=1, keepdims=True)
        csum = jnp.sum(jnp.exp(cl - cmax), axis=1, keepdims=True)
        c_idx = _first_argmax(cl, cmax, N_CLUSTERS)
        tl = lax.dot_general(xb, p_ref[...], (((1,), (1,)), ((), ())),
                             preferred_element_type=_F32)
        tcol = (lax.broadcasted_iota(jnp.int32, tl.shape, 1)
                // TILES_PER_CLUSTER)
        tlm = jnp.where(tcol == c_idx, tl, _NEG)
        tmax = jnp.max(tlm, axis=1, keepdims=True)
        tsum = jnp.sum(jnp.exp(tlm - tmax), axis=1, keepdims=True)
        tidx_ref[...] = _first_argmax(tlm, tmax, NUM_TILES)
        gate_ref[...] = 1.0 / (csum * tsum)
        xh_ref[...] = (xb * (1.0 / (D_MODEL ** 0.5))).astype(jnp.bfloat16)

    xh = xh_ref[...]
    kc = k_ref[...].astype(jnp.bfloat16)
    gl = lax.dot_general(xh, kc, (((1,), (1,)), ((), ())),
                         preferred_element_type=_F32)
    tcol = (lax.broadcasted_iota(jnp.int32, gl.shape, 1) // GRID_SIZE
            + c * TILES_PER_CHUNK)
    pr = jnp.where(tcol == tidx_ref[...],
                   jnp.exp(jnp.minimum(gl, 60.0)), 0.0)
    s = jnp.sum(pr, axis=1, keepdims=True)
    y = lax.dot_general(pr.astype(jnp.bfloat16), v_ref[...].astype(jnp.bfloat16),
                        (((1,), (0,)), ((), ())),
                        preferred_element_type=_F32)

    @pl.when(c == 0)
    def _init_acc():
        yacc_ref[...] = y
        sacc_ref[...] = s

    @pl.when(c > 0)
    def _accum():
        yacc_ref[...] += y
        sacc_ref[...] += s

    @pl.when(c == N_CHUNKS - 1)
    def _finalize():
        o_ref[...] = (x_ref[...]
                      + (gate_ref[...] / jnp.maximum(sacc_ref[...], 1e-30))
                      * yacc_ref[...])


@jax.jit
def kernel(x, Wc, P, Kt, Vt):
    n, d = x.shape
    k2 = Kt.reshape(TG, d)
    v2 = Vt.reshape(TG, d)
    return pl.pallas_call(
        _body,
        grid=(N_CHUNKS,),
        in_specs=[
            pl.BlockSpec((n, d), lambda c: (0, 0)),
            pl.BlockSpec((N_CLUSTERS, d), lambda c: (0, 0)),
            pl.BlockSpec((NUM_TILES, d), lambda c: (0, 0)),
            pl.BlockSpec((CHUNK, d), lambda c: (c, 0)),
            pl.BlockSpec((CHUNK, d), lambda c: (c, 0)),
        ],
        out_specs=pl.BlockSpec((n, d), lambda c: (0, 0)),
        out_shape=jax.ShapeDtypeStruct((n, d), jnp.float32),
        scratch_shapes=[
            pltpu.VMEM((n, d), jnp.bfloat16),
            pltpu.VMEM((n, 1), jnp.int32),
            pltpu.VMEM((n, 1), jnp.float32),
            pltpu.VMEM((n, d), jnp.float32),
            pltpu.VMEM((n, 1), jnp.float32),
        ],
        compiler_params=pltpu.CompilerParams(
            dimension_semantics=("arbitrary",),
        ),
    )(x, Wc, P, k2, v2)


# speedup vs baseline: 1.5135x; 1.5135x over previous
import jax
import jax.numpy as jnp
from jax import lax
from jax.experimental import pallas as pl
from jax.experimental.pallas import tpu as pltpu

D_MODEL = 1024
NUM_TILES = 64
TILES_PER_CLUSTER = 8
GRID_SIZE = 64
N_CLUSTERS = NUM_TILES // TILES_PER_CLUSTER
TG = NUM_TILES * GRID_SIZE
N_CHUNKS = 8
CHUNK = TG // N_CHUNKS
TILES_PER_CHUNK = CHUNK // GRID_SIZE

_NEG = -1e30
_F32 = jnp.float32


def _first_argmax(vals, maxv, width):
    col = lax.broadcasted_iota(jnp.int32, vals.shape, 1)
    cand = jnp.where(vals >= maxv, col, jnp.int32(width))
    return jnp.min(cand, axis=1, keepdims=True)


def _body(x_ref, wc_ref, p_ref, k_ref, v_ref, o_ref,
          xh_ref, tidx_ref, gate_ref, yacc_ref, sacc_ref):
    c = pl.program_id(0)

    @pl.when(c == 0)
    def _routing():
        xb = x_ref[...]
        cl = lax.dot_general(xb, wc_ref[...], (((1,), (1,)), ((), ())),
                             preferred_element_type=_F32)
        cmax = jnp.max(cl, axis=1, keepdims=True)
        csum = jnp.sum(jnp.exp(cl - cmax), axis=1, keepdims=True)
        c_idx = _first_argmax(cl, cmax, N_CLUSTERS)
        tl = lax.dot_general(xb, p_ref[...], (((1,), (1,)), ((), ())),
                             preferred_element_type=_F32)
        tcol = (lax.broadcasted_iota(jnp.int32, tl.shape, 1)
                // TILES_PER_CLUSTER)
        tlm = jnp.where(tcol == c_idx, tl, _NEG)
        tmax = jnp.max(tlm, axis=1, keepdims=True)
        tsum = jnp.sum(jnp.exp(tlm - tmax), axis=1, keepdims=True)
        tidx_ref[...] = _first_argmax(tlm, tmax, NUM_TILES)
        gate_ref[...] = 1.0 / (csum * tsum)
        xh_ref[...] = (xb * (1.0 / (D_MODEL ** 0.5))).astype(jnp.bfloat16)

    xh = xh_ref[...]
    kc = k_ref[...].astype(jnp.bfloat16)
    gl = lax.dot_general(xh, kc, (((1,), (1,)), ((), ())),
                         preferred_element_type=_F32)
    tcol = (lax.broadcasted_iota(jnp.int32, gl.shape, 1) // GRID_SIZE
            + c * TILES_PER_CHUNK)
    pr = jnp.where(tcol == tidx_ref[...],
                   jnp.exp(jnp.minimum(gl, 60.0)), 0.0)
    s = jnp.sum(pr, axis=1, keepdims=True)
    y = lax.dot_general(pr.astype(jnp.bfloat16), v_ref[...].astype(jnp.bfloat16),
                        (((1,), (0,)), ((), ())),
                        preferred_element_type=_F32)

    @pl.when(c == 0)
    def _init_acc():
        yacc_ref[...] = y
        sacc_ref[...] = s

    @pl.when(c > 0)
    def _accum():
        yacc_ref[...] += y
        sacc_ref[...] += s

    @pl.when(c == N_CHUNKS - 1)
    def _finalize():
        o_ref[...] = (x_ref[...]
                      + (gate_ref[...] / jnp.maximum(sacc_ref[...], 1e-30))
                      * yacc_ref[...])


@jax.jit
def kernel(x, Wc, P, Kt, Vt):
    n, d = x.shape
    k2 = Kt.reshape(TG, d)
    v2 = Vt.reshape(TG, d)
    return pl.pallas_call(
        _body,
        grid=(N_CHUNKS,),
        in_specs=[
            pl.BlockSpec((n, d), lambda c: (0, 0)),
            pl.BlockSpec((N_CLUSTERS, d), lambda c: (0, 0)),
            pl.BlockSpec((NUM_TILES, d), lambda c: (0, 0)),
            pl.BlockSpec((CHUNK, d), lambda c: (c, 0)),
            pl.BlockSpec((CHUNK, d), lambda c: (c, 0)),
        ],
        out_specs=pl.BlockSpec((n, d), lambda c: (0, 0)),
        out_shape=jax.ShapeDtypeStruct((n, d), jnp.float32),
        scratch_shapes=[
            pltpu.VMEM((n, d), jnp.bfloat16),
            pltpu.VMEM((n, 1), jnp.int32),
            pltpu.VMEM((n, 1), jnp.float32),
            pltpu.VMEM((n, d), jnp.float32),
            pltpu.VMEM((n, 1), jnp.float32),
        ],
        compiler_params=pltpu.CompilerParams(
            dimension_semantics=("arbitrary",),
        ),
    )(x, Wc, P, k2, v2)
